# Initial kernel scaffold; baseline (speedup 1.0000x reference)
import jax
import jax.numpy as jnp
from jax.experimental import pallas as pl


def kernel(inputs, word_table, pos_table):
    raise NotImplementedError("write your pallas kernel here")


# speedup vs baseline: 2.3227x; 2.3227x over previous
import functools

import jax
import jax.numpy as jnp
from jax import lax
from jax.experimental import pallas as pl
from jax.experimental.pallas import tpu as pltpu
from jax.experimental.pallas import tpu_sc as plsc

SEQ = 200
D = 64
BATCH = 1024

CHUNKS = ((0, 128), (128, 72))
NC, NS = 2, 16
NW = NC * NS
SEQ_PER_W = BATCH // NW
TOTAL_ROWS = BATCH * SEQ


def _make_kernel():
    mesh = plsc.VectorSubcoreMesh(core_axis_name="c", subcore_axis_name="s")

    @functools.partial(
        pl.kernel,
        out_type=jax.ShapeDtypeStruct((TOTAL_ROWS, D), jnp.float32),
        mesh=mesh,
        compiler_params=pltpu.CompilerParams(use_tc_tiling_on_sc=False),
        scratch_types=[
            pltpu.VMEM((128,), jnp.int32),
            pltpu.VMEM((72,), jnp.int32),
            pltpu.VMEM((128, D), jnp.float32),
            pltpu.VMEM((72, D), jnp.float32),
            pltpu.VMEM((SEQ, D), jnp.float32),
            pltpu.SemaphoreType.DMA,
        ],
    )
    def k(idx_hbm, word_hbm, pos_hbm, out_hbm,
          idx_a, idx_b, rows_a, rows_b, pos_v, sem):
        wid = lax.axis_index("s") * NC + lax.axis_index("c")
        pltpu.sync_copy(pos_hbm, pos_v)

        def seq_body(s, _):
            base = (wid * SEQ_PER_W + s) * SEQ
            for (off, n), idx_v, rows_v in zip(CHUNKS, (idx_a, idx_b),
                                               (rows_a, rows_b)):
                pltpu.sync_copy(idx_hbm.at[pl.ds(base + off, n)], idx_v)
                pltpu.async_copy(word_hbm.at[idx_v], rows_v, sem).wait()

                def add_row(i, _):
                    for c in range(D // 16):
                        sl = pl.ds(c * 16, 16)
                        rows_v[i, sl] = rows_v[i, sl] + pos_v[off + i, sl]
                    return 0

                lax.fori_loop(0, n, add_row, 0)
                pltpu.sync_copy(rows_v, out_hbm.at[pl.ds(base + off, n)])
            return 0

        lax.fori_loop(0, SEQ_PER_W, seq_body, 0)

    return k


_kernel = _make_kernel()


@jax.jit
def kernel(inputs, word_table, pos_table):
    idx = inputs.astype(jnp.int32).reshape(TOTAL_ROWS)
    out = _kernel(idx, word_table, pos_table)
    return out.reshape(BATCH, SEQ, D)
